# baseline (speedup 1.0000x reference)
import functools

import jax
import jax.numpy as jnp
from jax import lax
from jax.experimental import pallas as pl
from jax.experimental.pallas import tpu as pltpu
from jax.experimental.pallas import tpu_sc as plsc

E = 8
D = 1024
F = 2048
K = 2
T = 2048

BLK = 256
NBLK = 24
SLOTS = NBLK * BLK
NC, NS = 2, 16
NW = NC * NS
TPW = T // NW
HALF = TPW // 2
FC = 512
TB = 256


def _gate_body(x_ref, wg_ref, bg_ref, i1_ref, i2_ref, g1_ref, g2_ref):
    l = jnp.dot(x_ref[...], wg_ref[...], preferred_element_type=jnp.float32)
    l = l + bg_ref[0, :]
    iot = lax.broadcasted_iota(jnp.int32, l.shape, 1)
    m1 = jnp.max(l, axis=1, keepdims=True)
    i1 = jnp.min(jnp.where(l == m1, iot, E), axis=1, keepdims=True)
    l2 = jnp.where(iot == i1, -jnp.inf, l)
    m2 = jnp.max(l2, axis=1, keepdims=True)
    i2 = jnp.min(jnp.where(l2 == m2, iot, E), axis=1, keepdims=True)
    s1 = 1.0 / (1.0 + jnp.exp(m2 - m1))
    i1_ref[...] = jnp.broadcast_to(i1, i1_ref.shape)
    i2_ref[...] = jnp.broadcast_to(i2, i2_ref.shape)
    g1_ref[...] = jnp.broadcast_to(s1, g1_ref.shape)
    g2_ref[...] = jnp.broadcast_to(1.0 - s1, g2_ref.shape)


_gate_call = pl.pallas_call(
    _gate_body,
    grid=(T // TB,),
    in_specs=[
        pl.BlockSpec((TB, D), lambda i: (i, 0)),
        pl.BlockSpec((D, E), lambda i: (0, 0)),
        pl.BlockSpec((1, E), lambda i: (0, 0)),
    ],
    out_specs=[
        pl.BlockSpec((TB, E), lambda i: (i, 0)),
        pl.BlockSpec((TB, E), lambda i: (i, 0)),
        pl.BlockSpec((TB, 16), lambda i: (i, 0)),
        pl.BlockSpec((TB, 16), lambda i: (i, 0)),
    ],
    out_shape=[
        jax.ShapeDtypeStruct((T, E), jnp.int32),
        jax.ShapeDtypeStruct((T, E), jnp.int32),
        jax.ShapeDtypeStruct((T, 16), jnp.float32),
        jax.ShapeDtypeStruct((T, 16), jnp.float32),
    ],
)


def _dispatch_body(x_hbm, d0_hbm, d1_hbm, xs_hbm, i0_v, i1_v, rows_v, sem):
    w = lax.axis_index("s") * NC + lax.axis_index("c")
    pltpu.sync_copy(x_hbm.at[pl.ds(w * TPW, TPW)], rows_v)
    pltpu.sync_copy(d0_hbm.at[w], i0_v)
    pltpu.sync_copy(d1_hbm.at[w], i1_v)
    pltpu.async_copy(rows_v, xs_hbm.at[i0_v], sem).wait()
    pltpu.async_copy(rows_v, xs_hbm.at[i1_v], sem).wait()


@functools.cache
def _dispatch_call():
    return pl.kernel(
        _dispatch_body,
        out_type=jax.ShapeDtypeStruct((SLOTS, D), jnp.float32),
        mesh=plsc.VectorSubcoreMesh(core_axis_name="c", subcore_axis_name="s"),
        scratch_types=[
            pltpu.VMEM((TPW,), jnp.int32),
            pltpu.VMEM((TPW,), jnp.int32),
            pltpu.VMEM((TPW, D), jnp.float32),
            pltpu.SemaphoreType.DMA,
        ],
    )


def _ffn_body(emap, act, xs_ref, w1_ref, b1_ref, w2_ref, b2_ref, out_ref):
    b = pl.program_id(0)
    f = pl.program_id(1)

    @pl.when(act[b] == 1)
    def _():
        x = xs_ref[...]
        h = jnp.dot(x, w1_ref[0], preferred_element_type=jnp.float32)
        h = jnp.maximum(h + b1_ref[0, 0, :], 0.0)
        p = jnp.dot(h, w2_ref[0], preferred_element_type=jnp.float32)

        @pl.when(f == 0)
        def _():
            out_ref[...] = p + b2_ref[0, 0, :]

        @pl.when(f != 0)
        def _():
            out_ref[...] += p


_ffn_call = pl.pallas_call(
    _ffn_body,
    grid_spec=pltpu.PrefetchScalarGridSpec(
        num_scalar_prefetch=2,
        grid=(NBLK, F // FC),
        in_specs=[
            pl.BlockSpec((BLK, D), lambda b, f, em, ac: (b, 0)),
            pl.BlockSpec((1, D, FC), lambda b, f, em, ac: (em[b], 0, f)),
            pl.BlockSpec((1, 1, FC), lambda b, f, em, ac: (em[b], 0, f)),
            pl.BlockSpec((1, FC, D), lambda b, f, em, ac: (em[b], f, 0)),
            pl.BlockSpec((1, 1, D), lambda b, f, em, ac: (em[b], 0, 0)),
        ],
        out_specs=pl.BlockSpec((BLK, D), lambda b, f, em, ac: (b, 0)),
    ),
    out_shape=jax.ShapeDtypeStruct((SLOTS, D), jnp.float32),
)


def _combine_body(ys_hbm, d0_hbm, d1_hbm, g0_hbm, g1_hbm, out_hbm,
                  i0_v, i1_v, y0_v, y1_v, g0_v, g1_v, ob_v, sem):
    w = lax.axis_index("s") * NC + lax.axis_index("c")
    for hh in range(TPW // HALF):
        t0 = w * TPW + hh * HALF
        pltpu.sync_copy(d0_hbm.at[w, pl.ds(hh * HALF, HALF)], i0_v)
        pltpu.sync_copy(d1_hbm.at[w, pl.ds(hh * HALF, HALF)], i1_v)
        pltpu.sync_copy(g0_hbm.at[pl.ds(t0, HALF)], g0_v)
        pltpu.sync_copy(g1_hbm.at[pl.ds(t0, HALF)], g1_v)
        pltpu.async_copy(ys_hbm.at[i0_v], y0_v, sem).wait()
        pltpu.async_copy(ys_hbm.at[i1_v], y1_v, sem).wait()

        def tok(j, carry):
            a = g0_v[j, :]
            bb = g1_v[j, :]
            for v in range(D // 16):
                sl = pl.ds(v * 16, 16)
                ob_v[j, sl] = a * y0_v[j, sl] + bb * y1_v[j, sl]
            return carry

        lax.fori_loop(0, HALF, tok, 0)
        pltpu.sync_copy(ob_v, out_hbm.at[pl.ds(t0, HALF)])


@functools.cache
def _combine_call():
    return pl.kernel(
        _combine_body,
        out_type=jax.ShapeDtypeStruct((T, D), jnp.float32),
        mesh=plsc.VectorSubcoreMesh(core_axis_name="c", subcore_axis_name="s"),
        scratch_types=[
            pltpu.VMEM((HALF,), jnp.int32),
            pltpu.VMEM((HALF,), jnp.int32),
            pltpu.VMEM((HALF, D), jnp.float32),
            pltpu.VMEM((HALF, D), jnp.float32),
            pltpu.VMEM((HALF, 16), jnp.float32),
            pltpu.VMEM((HALF, 16), jnp.float32),
            pltpu.VMEM((HALF, D), jnp.float32),
            pltpu.SemaphoreType.DMA,
        ],
    )


def kernel(moe_inp, original_shape, total_experts, top_k, layer_idx,
           Wg, bg, W1, b1, W2, b2):
    x = moe_inp
    i1b, i2b, g1r, g2r = _gate_call(x, Wg, bg.reshape(1, E))
    i1 = i1b[:, 0]
    i2 = i2b[:, 0]

    flat = jnp.stack([i1, i2], axis=1).reshape(-1)
    oh = (flat[:, None] == jnp.arange(E, dtype=flat.dtype)[None, :])
    oh = oh.astype(jnp.int32)
    cnt = jnp.sum(oh, axis=0)
    padc = ((cnt + (BLK - 1)) // BLK) * BLK
    ends = jnp.cumsum(padc)
    offs = ends - padc
    rank = jnp.cumsum(oh, axis=0) - oh
    r = jnp.take_along_axis(rank, flat[:, None], axis=1)[:, 0]
    dest = (offs[flat] + r).astype(jnp.int32)
    dest2 = dest.reshape(T, K)
    d0 = dest2[:, 0].reshape(NW, TPW)
    d1 = dest2[:, 1].reshape(NW, TPW)

    bs = jnp.arange(NBLK, dtype=jnp.int32) * BLK
    eb = jnp.searchsorted(ends, bs, side="right").astype(jnp.int32)
    emap = jnp.minimum(eb, E - 1)
    act = ((eb < E) & (bs < offs[emap] + cnt[emap])).astype(jnp.int32)

    xs = _dispatch_call()(x, d0, d1)
    ys = _ffn_call(emap, act, xs, W1, b1.reshape(E, 1, F),
                   W2, b2.reshape(E, 1, D))
    out = _combine_call()(ys, d0, d1, g1r, g2r)
    return out


# speedup vs baseline: 1.4574x; 1.4574x over previous
import functools

import jax
import jax.numpy as jnp
from jax import lax
from jax.experimental import pallas as pl
from jax.experimental.pallas import tpu as pltpu
from jax.experimental.pallas import tpu_sc as plsc

E = 8
D = 1024
F = 2048
K = 2
T = 2048

BLK = 256
NBLK = 24
SLOTS = NBLK * BLK
NC, NS = 2, 16
NW = NC * NS
TPW = T // NW
HALF = TPW // 2
FC = 512
TB = 256


NTB = T // TB
_HIGH = jax.lax.Precision.HIGHEST


def _gate_body(x_ref, wg_ref, bg_ref,
               d0_ref, d1_ref, g1_ref, g2_ref, meta_ref,
               si1, si2, ss1, scnt, srun, soff):
    p = pl.program_id(0)
    t = pl.program_id(1)
    iota8 = lax.broadcasted_iota(jnp.int32, (TB, E), 1)

    @pl.when(p == 0)
    def _pass0():
        l = jnp.dot(x_ref[...], wg_ref[...], preferred_element_type=jnp.float32)
        l = l + bg_ref[0, :]
        m1 = jnp.max(l, axis=1, keepdims=True)
        i1 = jnp.min(jnp.where(l == m1, iota8, E), axis=1, keepdims=True)
        l2 = jnp.where(iota8 == i1, -jnp.inf, l)
        m2 = jnp.max(l2, axis=1, keepdims=True)
        i2 = jnp.min(jnp.where(l2 == m2, iota8, E), axis=1, keepdims=True)
        s1 = 1.0 / (1.0 + jnp.exp(m2 - m1))
        sl = pl.ds(t * TB, TB)
        si1[sl, :] = jnp.broadcast_to(i1, (TB, E))
        si2[sl, :] = jnp.broadcast_to(i2, (TB, E))
        ss1[sl, :] = jnp.broadcast_to(s1, (TB, E))

        @pl.when(t == 0)
        def _():
            scnt[...] = jnp.zeros((1, E), jnp.float32)

        oh = ((i1 == iota8) | (i2 == iota8)).astype(jnp.float32)
        scnt[...] += jnp.sum(oh, axis=0, keepdims=True)

    @pl.when(p == 1)
    def _pass1():
        @pl.when(t == 0)
        def _():
            cnt = scnt[...]
            cnti = cnt.astype(jnp.int32)
            padc = (((cnti + (BLK - 1)) // BLK) * BLK).astype(jnp.float32)
            r8 = lax.broadcasted_iota(jnp.int32, (E, E), 0)
            c8 = lax.broadcasted_iota(jnp.int32, (E, E), 1)
            U = (r8 <= c8).astype(jnp.float32)
            P88 = jnp.broadcast_to(padc, (E, E))
            ends = jnp.dot(P88, U, precision=_HIGH)[0:1, :]
            offs = ends - padc
            soff[...] = offs
            srun[...] = jnp.zeros((1, E), jnp.float32)
            eye = (r8 == c8).astype(jnp.float32)
            tdot = lambda a, b: lax.dot_general(
                a, b, (((1,), (1,)), ((), ())), precision=_HIGH)
            ends_c = tdot(eye, ends)
            offs_c = tdot(eye, offs)
            cnt_c = tdot(eye, cnt)
            bs = (lax.broadcasted_iota(jnp.int32, (E, 64), 1) * BLK
                  ).astype(jnp.float32)
            emap = jnp.sum((jnp.broadcast_to(ends_c, (E, 64)) <= bs)
                           .astype(jnp.int32), axis=0, keepdims=True)
            emap = jnp.minimum(emap, E - 1)
            inseg = ((bs >= offs_c) & (bs < offs_c + cnt_c)).astype(jnp.int32)
            act = jnp.sum(inseg, axis=0, keepdims=True)
            meta_ref[...] = jnp.concatenate([emap, act], axis=0)

        sl = pl.ds(t * TB, TB)
        i1 = si1[sl, :]
        i2 = si2[sl, :]
        oh1 = (i1 == iota8).astype(jnp.float32)
        oh2 = (i2 == iota8).astype(jnp.float32)
        oh = oh1 + oh2
        rr = lax.broadcasted_iota(jnp.int32, (TB, TB), 0)
        cc = lax.broadcasted_iota(jnp.int32, (TB, TB), 1)
        S = (cc < rr).astype(jnp.float32)
        C = jnp.dot(S, oh, precision=_HIGH)
        A = C + soff[...] + srun[...]
        d0_ref[...] = jnp.sum(oh1 * A, axis=1, keepdims=True).astype(jnp.int32)
        d1_ref[...] = jnp.sum(oh2 * A, axis=1, keepdims=True).astype(jnp.int32)
        s1 = ss1[sl, 0:1]
        g1_ref[...] = jnp.broadcast_to(s1, (TB, 16))
        g2_ref[...] = jnp.broadcast_to(1.0 - s1, (TB, 16))
        srun[...] += jnp.sum(oh, axis=0, keepdims=True)


_gate_call = pl.pallas_call(
    _gate_body,
    grid=(2, NTB),
    in_specs=[
        pl.BlockSpec((TB, D), lambda p, t: (jnp.where(p == 0, t, NTB - 1), 0)),
        pl.BlockSpec((D, E), lambda p, t: (0, 0)),
        pl.BlockSpec((1, E), lambda p, t: (0, 0)),
    ],
    out_specs=[
        pl.BlockSpec((TB, 1), lambda p, t: (t, 0)),
        pl.BlockSpec((TB, 1), lambda p, t: (t, 0)),
        pl.BlockSpec((TB, 16), lambda p, t: (t, 0)),
        pl.BlockSpec((TB, 16), lambda p, t: (t, 0)),
        pl.BlockSpec((2, 64), lambda p, t: (0, 0)),
    ],
    out_shape=[
        jax.ShapeDtypeStruct((T, 1), jnp.int32),
        jax.ShapeDtypeStruct((T, 1), jnp.int32),
        jax.ShapeDtypeStruct((T, 16), jnp.float32),
        jax.ShapeDtypeStruct((T, 16), jnp.float32),
        jax.ShapeDtypeStruct((2, 64), jnp.int32),
    ],
    scratch_shapes=[
        pltpu.VMEM((T, E), jnp.int32),
        pltpu.VMEM((T, E), jnp.int32),
        pltpu.VMEM((T, E), jnp.float32),
        pltpu.VMEM((1, E), jnp.float32),
        pltpu.VMEM((1, E), jnp.float32),
        pltpu.VMEM((1, E), jnp.float32),
    ],
)


def _dispatch_body(x_hbm, d0_hbm, d1_hbm, xs_hbm, i0_v, i1_v, rows_v, sem):
    w = lax.axis_index("s") * NC + lax.axis_index("c")
    pltpu.sync_copy(x_hbm.at[pl.ds(w * TPW, TPW)], rows_v)
    pltpu.sync_copy(d0_hbm.at[w], i0_v)
    pltpu.sync_copy(d1_hbm.at[w], i1_v)
    pltpu.async_copy(rows_v, xs_hbm.at[i0_v], sem).wait()
    pltpu.async_copy(rows_v, xs_hbm.at[i1_v], sem).wait()


@functools.cache
def _dispatch_call():
    return pl.kernel(
        _dispatch_body,
        out_type=jax.ShapeDtypeStruct((SLOTS, D), jnp.float32),
        mesh=plsc.VectorSubcoreMesh(core_axis_name="c", subcore_axis_name="s"),
        scratch_types=[
            pltpu.VMEM((TPW,), jnp.int32),
            pltpu.VMEM((TPW,), jnp.int32),
            pltpu.VMEM((TPW, D), jnp.float32),
            pltpu.SemaphoreType.DMA,
        ],
    )


def _ffn_body(emap, act, xs_ref, w1_ref, b1_ref, w2_ref, b2_ref, out_ref):
    b = pl.program_id(0)

    @pl.when(act[b] == 1)
    def _():
        x = xs_ref[...]
        for f in range(F // FC):
            sl = slice(f * FC, (f + 1) * FC)
            h = jnp.dot(x, w1_ref[0][:, sl], preferred_element_type=jnp.float32)
            h = jnp.maximum(h + b1_ref[0, 0, sl], 0.0)
            p = jnp.dot(h, w2_ref[0][sl, :], preferred_element_type=jnp.float32)
            if f == 0:
                out_ref[...] = p + b2_ref[0, 0, :]
            else:
                out_ref[...] += p


_ffn_call = pl.pallas_call(
    _ffn_body,
    grid_spec=pltpu.PrefetchScalarGridSpec(
        num_scalar_prefetch=2,
        grid=(NBLK,),
        in_specs=[
            pl.BlockSpec((BLK, D), lambda b, em, ac: (b, 0)),
            pl.BlockSpec((1, D, F), lambda b, em, ac: (em[b], 0, 0)),
            pl.BlockSpec((1, 1, F), lambda b, em, ac: (em[b], 0, 0)),
            pl.BlockSpec((1, F, D), lambda b, em, ac: (em[b], 0, 0)),
            pl.BlockSpec((1, 1, D), lambda b, em, ac: (em[b], 0, 0)),
        ],
        out_specs=pl.BlockSpec((BLK, D), lambda b, em, ac: (b, 0)),
    ),
    out_shape=jax.ShapeDtypeStruct((SLOTS, D), jnp.float32),
)


def _combine_body(ys_hbm, d0_hbm, d1_hbm, g0_hbm, g1_hbm, out_hbm,
                  i0_v, i1_v, y0_v, y1_v, g0_v, g1_v, ob_v, sem):
    w = lax.axis_index("s") * NC + lax.axis_index("c")
    for hh in range(TPW // HALF):
        t0 = w * TPW + hh * HALF
        pltpu.sync_copy(d0_hbm.at[w, pl.ds(hh * HALF, HALF)], i0_v)
        pltpu.sync_copy(d1_hbm.at[w, pl.ds(hh * HALF, HALF)], i1_v)
        pltpu.sync_copy(g0_hbm.at[pl.ds(t0, HALF)], g0_v)
        pltpu.sync_copy(g1_hbm.at[pl.ds(t0, HALF)], g1_v)
        pltpu.async_copy(ys_hbm.at[i0_v], y0_v, sem).wait()
        pltpu.async_copy(ys_hbm.at[i1_v], y1_v, sem).wait()

        def tok(j, carry):
            a = g0_v[j, :]
            bb = g1_v[j, :]
            for v in range(D // 16):
                sl = pl.ds(v * 16, 16)
                ob_v[j, sl] = a * y0_v[j, sl] + bb * y1_v[j, sl]
            return carry

        lax.fori_loop(0, HALF, tok, 0)
        pltpu.sync_copy(ob_v, out_hbm.at[pl.ds(t0, HALF)])


@functools.cache
def _combine_call():
    return pl.kernel(
        _combine_body,
        out_type=jax.ShapeDtypeStruct((T, D), jnp.float32),
        mesh=plsc.VectorSubcoreMesh(core_axis_name="c", subcore_axis_name="s"),
        scratch_types=[
            pltpu.VMEM((HALF,), jnp.int32),
            pltpu.VMEM((HALF,), jnp.int32),
            pltpu.VMEM((HALF, D), jnp.float32),
            pltpu.VMEM((HALF, D), jnp.float32),
            pltpu.VMEM((HALF, 16), jnp.float32),
            pltpu.VMEM((HALF, 16), jnp.float32),
            pltpu.VMEM((HALF, D), jnp.float32),
            pltpu.SemaphoreType.DMA,
        ],
    )


def kernel(moe_inp, original_shape, total_experts, top_k, layer_idx,
           Wg, bg, W1, b1, W2, b2):
    x = moe_inp
    d0c, d1c, g1r, g2r, meta = _gate_call(x, Wg, bg.reshape(1, E))
    d0 = d0c.reshape(NW, TPW)
    d1 = d1c.reshape(NW, TPW)
    emap = meta[0, :NBLK]
    act = meta[1, :NBLK]

    xs = _dispatch_call()(x, d0, d1)
    ys = _ffn_call(emap, act, xs, W1, b1.reshape(E, 1, F),
                   W2, b2.reshape(E, 1, D))
    out = _combine_call()(ys, d0, d1, g1r, g2r)
    return out


# speedup vs baseline: 1.5828x; 1.0860x over previous
import functools

import jax
import jax.numpy as jnp
from jax import lax
from jax.experimental import pallas as pl
from jax.experimental.pallas import tpu as pltpu
from jax.experimental.pallas import tpu_sc as plsc

E = 8
D = 1024
F = 2048
K = 2
T = 2048

BLK = 512
NBLK = 16
SLOTS = NBLK * BLK
NC, NS = 2, 16
NW = NC * NS
TPW = T // NW
HALF = TPW // 2
FC = 512
TB = 256


NTB = T // TB
_HIGH = jax.lax.Precision.HIGHEST


def _gate_body(x_ref, wg_ref, bg_ref,
               d0_ref, d1_ref, g1_ref, g2_ref, meta_ref,
               si1, si2, ss1, scnt, srun, soff):
    p = pl.program_id(0)
    t = pl.program_id(1)
    iota8 = lax.broadcasted_iota(jnp.int32, (TB, E), 1)

    @pl.when(p == 0)
    def _pass0():
        l = jnp.dot(x_ref[...], wg_ref[...], preferred_element_type=jnp.float32)
        l = l + bg_ref[0, :]
        m1 = jnp.max(l, axis=1, keepdims=True)
        i1 = jnp.min(jnp.where(l == m1, iota8, E), axis=1, keepdims=True)
        l2 = jnp.where(iota8 == i1, -jnp.inf, l)
        m2 = jnp.max(l2, axis=1, keepdims=True)
        i2 = jnp.min(jnp.where(l2 == m2, iota8, E), axis=1, keepdims=True)
        s1 = 1.0 / (1.0 + jnp.exp(m2 - m1))
        sl = pl.ds(t * TB, TB)
        si1[sl, :] = jnp.broadcast_to(i1, (TB, E))
        si2[sl, :] = jnp.broadcast_to(i2, (TB, E))
        ss1[sl, :] = jnp.broadcast_to(s1, (TB, E))

        @pl.when(t == 0)
        def _():
            scnt[...] = jnp.zeros((1, E), jnp.float32)

        oh = ((i1 == iota8) | (i2 == iota8)).astype(jnp.float32)
        scnt[...] += jnp.sum(oh, axis=0, keepdims=True)

    @pl.when(p == 1)
    def _pass1():
        @pl.when(t == 0)
        def _():
            cnt = scnt[...]
            cnti = cnt.astype(jnp.int32)
            padc = (((cnti + (BLK - 1)) // BLK) * BLK).astype(jnp.float32)
            r8 = lax.broadcasted_iota(jnp.int32, (E, E), 0)
            c8 = lax.broadcasted_iota(jnp.int32, (E, E), 1)
            U = (r8 <= c8).astype(jnp.float32)
            P88 = jnp.broadcast_to(padc, (E, E))
            ends = jnp.dot(P88, U, precision=_HIGH)[0:1, :]
            offs = ends - padc
            soff[...] = offs
            srun[...] = jnp.zeros((1, E), jnp.float32)
            eye = (r8 == c8).astype(jnp.float32)
            tdot = lambda a, b: lax.dot_general(
                a, b, (((1,), (1,)), ((), ())), precision=_HIGH)
            ends_c = tdot(eye, ends)
            offs_c = tdot(eye, offs)
            cnt_c = tdot(eye, cnt)
            bs = (lax.broadcasted_iota(jnp.int32, (E, 64), 1) * BLK
                  ).astype(jnp.float32)
            emap = jnp.sum((jnp.broadcast_to(ends_c, (E, 64)) <= bs)
                           .astype(jnp.int32), axis=0, keepdims=True)
            emap = jnp.minimum(emap, E - 1)
            inseg = ((bs >= offs_c) & (bs < offs_c + cnt_c)).astype(jnp.int32)
            act = jnp.sum(inseg, axis=0, keepdims=True)
            meta_ref[...] = jnp.concatenate([emap, act], axis=0)

        sl = pl.ds(t * TB, TB)
        i1 = si1[sl, :]
        i2 = si2[sl, :]
        oh1 = (i1 == iota8).astype(jnp.float32)
        oh2 = (i2 == iota8).astype(jnp.float32)
        oh = oh1 + oh2
        rr = lax.broadcasted_iota(jnp.int32, (TB, TB), 0)
        cc = lax.broadcasted_iota(jnp.int32, (TB, TB), 1)
        S = (cc < rr).astype(jnp.float32)
        C = jnp.dot(S, oh, precision=_HIGH)
        A = C + soff[...] + srun[...]
        d0_ref[...] = jnp.sum(oh1 * A, axis=1, keepdims=True).astype(jnp.int32)
        d1_ref[...] = jnp.sum(oh2 * A, axis=1, keepdims=True).astype(jnp.int32)
        s1 = ss1[sl, 0:1]
        g1_ref[...] = jnp.broadcast_to(s1, (TB, 16))
        g2_ref[...] = jnp.broadcast_to(1.0 - s1, (TB, 16))
        srun[...] += jnp.sum(oh, axis=0, keepdims=True)


_gate_call = pl.pallas_call(
    _gate_body,
    grid=(2, NTB),
    in_specs=[
        pl.BlockSpec((TB, D), lambda p, t: (jnp.where(p == 0, t, NTB - 1), 0)),
        pl.BlockSpec((D, E), lambda p, t: (0, 0)),
        pl.BlockSpec((1, E), lambda p, t: (0, 0)),
    ],
    out_specs=[
        pl.BlockSpec((TB, 1), lambda p, t: (t, 0)),
        pl.BlockSpec((TB, 1), lambda p, t: (t, 0)),
        pl.BlockSpec((TB, 16), lambda p, t: (t, 0)),
        pl.BlockSpec((TB, 16), lambda p, t: (t, 0)),
        pl.BlockSpec((2, 64), lambda p, t: (0, 0)),
    ],
    out_shape=[
        jax.ShapeDtypeStruct((T, 1), jnp.int32),
        jax.ShapeDtypeStruct((T, 1), jnp.int32),
        jax.ShapeDtypeStruct((T, 16), jnp.float32),
        jax.ShapeDtypeStruct((T, 16), jnp.float32),
        jax.ShapeDtypeStruct((2, 64), jnp.int32),
    ],
    scratch_shapes=[
        pltpu.VMEM((T, E), jnp.int32),
        pltpu.VMEM((T, E), jnp.int32),
        pltpu.VMEM((T, E), jnp.float32),
        pltpu.VMEM((1, E), jnp.float32),
        pltpu.VMEM((1, E), jnp.float32),
        pltpu.VMEM((1, E), jnp.float32),
    ],
)


def _dispatch_body(x_hbm, d0_hbm, d1_hbm, xs_hbm, i0_v, i1_v, rows_v, sem):
    w = lax.axis_index("s") * NC + lax.axis_index("c")
    pltpu.sync_copy(x_hbm.at[pl.ds(w * TPW, TPW)], rows_v)
    pltpu.sync_copy(d0_hbm.at[w], i0_v)
    pltpu.sync_copy(d1_hbm.at[w], i1_v)
    pltpu.async_copy(rows_v, xs_hbm.at[i0_v], sem).wait()
    pltpu.async_copy(rows_v, xs_hbm.at[i1_v], sem).wait()


@functools.cache
def _dispatch_call():
    return pl.kernel(
        _dispatch_body,
        out_type=jax.ShapeDtypeStruct((SLOTS, D), jnp.float32),
        mesh=plsc.VectorSubcoreMesh(core_axis_name="c", subcore_axis_name="s"),
        scratch_types=[
            pltpu.VMEM((TPW,), jnp.int32),
            pltpu.VMEM((TPW,), jnp.int32),
            pltpu.VMEM((TPW, D), jnp.float32),
            pltpu.SemaphoreType.DMA,
        ],
    )


def _ffn_body(emap, act, xs_ref, w1_ref, b1_ref, w2_ref, b2_ref, out_ref):
    b = pl.program_id(0)

    @pl.when(act[b] == 1)
    def _():
        x = xs_ref[...]
        for f in range(F // FC):
            sl = slice(f * FC, (f + 1) * FC)
            h = jnp.dot(x, w1_ref[0][:, sl], preferred_element_type=jnp.float32)
            h = jnp.maximum(h + b1_ref[0, 0, sl], 0.0)
            p = jnp.dot(h, w2_ref[0][sl, :], preferred_element_type=jnp.float32)
            if f == 0:
                out_ref[...] = p + b2_ref[0, 0, :]
            else:
                out_ref[...] += p


_ffn_call = pl.pallas_call(
    _ffn_body,
    grid_spec=pltpu.PrefetchScalarGridSpec(
        num_scalar_prefetch=2,
        grid=(NBLK,),
        in_specs=[
            pl.BlockSpec((BLK, D), lambda b, em, ac: (b, 0)),
            pl.BlockSpec((1, D, F), lambda b, em, ac: (em[b], 0, 0)),
            pl.BlockSpec((1, 1, F), lambda b, em, ac: (em[b], 0, 0)),
            pl.BlockSpec((1, F, D), lambda b, em, ac: (em[b], 0, 0)),
            pl.BlockSpec((1, 1, D), lambda b, em, ac: (em[b], 0, 0)),
        ],
        out_specs=pl.BlockSpec((BLK, D), lambda b, em, ac: (b, 0)),
    ),
    out_shape=jax.ShapeDtypeStruct((SLOTS, D), jnp.float32),
)


def _combine_body(ys_hbm, d0_hbm, d1_hbm, g0_hbm, g1_hbm, out_hbm,
                  i0_v, i1_v, y0_v, y1_v, g0_v, g1_v, ob_v, sem):
    w = lax.axis_index("s") * NC + lax.axis_index("c")
    for hh in range(TPW // HALF):
        t0 = w * TPW + hh * HALF
        pltpu.sync_copy(d0_hbm.at[w, pl.ds(hh * HALF, HALF)], i0_v)
        pltpu.sync_copy(d1_hbm.at[w, pl.ds(hh * HALF, HALF)], i1_v)
        pltpu.sync_copy(g0_hbm.at[pl.ds(t0, HALF)], g0_v)
        pltpu.sync_copy(g1_hbm.at[pl.ds(t0, HALF)], g1_v)
        pltpu.async_copy(ys_hbm.at[i0_v], y0_v, sem).wait()
        pltpu.async_copy(ys_hbm.at[i1_v], y1_v, sem).wait()

        def tok(j, carry):
            a = g0_v[j, :]
            bb = g1_v[j, :]
            for v in range(D // 16):
                sl = pl.ds(v * 16, 16)
                ob_v[j, sl] = a * y0_v[j, sl] + bb * y1_v[j, sl]
            return carry

        lax.fori_loop(0, HALF, tok, 0)
        pltpu.sync_copy(ob_v, out_hbm.at[pl.ds(t0, HALF)])


@functools.cache
def _combine_call():
    return pl.kernel(
        _combine_body,
        out_type=jax.ShapeDtypeStruct((T, D), jnp.float32),
        mesh=plsc.VectorSubcoreMesh(core_axis_name="c", subcore_axis_name="s"),
        scratch_types=[
            pltpu.VMEM((HALF,), jnp.int32),
            pltpu.VMEM((HALF,), jnp.int32),
            pltpu.VMEM((HALF, D), jnp.float32),
            pltpu.VMEM((HALF, D), jnp.float32),
            pltpu.VMEM((HALF, 16), jnp.float32),
            pltpu.VMEM((HALF, 16), jnp.float32),
            pltpu.VMEM((HALF, D), jnp.float32),
            pltpu.SemaphoreType.DMA,
        ],
    )


def kernel(moe_inp, original_shape, total_experts, top_k, layer_idx,
           Wg, bg, W1, b1, W2, b2):
    x = moe_inp
    d0c, d1c, g1r, g2r, meta = _gate_call(x, Wg, bg.reshape(1, E))
    d0 = d0c.reshape(NW, TPW)
    d1 = d1c.reshape(NW, TPW)
    emap = meta[0, :NBLK]
    act = meta[1, :NBLK]

    xs = _dispatch_call()(x, d0, d1)
    ys = _ffn_call(emap, act, xs, W1, b1.reshape(E, 1, F),
                   W2, b2.reshape(E, 1, D))
    out = _combine_call()(ys, d0, d1, g1r, g2r)
    return out


# speedup vs baseline: 1.6250x; 1.0267x over previous
import functools

import jax
import jax.numpy as jnp
from jax import lax
from jax.experimental import pallas as pl
from jax.experimental.pallas import tpu as pltpu
from jax.experimental.pallas import tpu_sc as plsc

E = 8
D = 1024
F = 2048
K = 2
T = 2048

BLK = 512
NBLK = 16
SLOTS = NBLK * BLK
NC, NS = 2, 16
NW = NC * NS
TPW = T // NW
HALF = TPW // 2
FC = 512
TB = 256


NTB = T // TB
_HIGH = jax.lax.Precision.HIGHEST


def _gate_body(x_ref, wg_ref, bg_ref,
               d0_ref, d1_ref, g1_ref, g2_ref, meta_ref,
               si1, si2, ss1, scnt, srun, soff):
    p = pl.program_id(0)
    t = pl.program_id(1)
    iota8 = lax.broadcasted_iota(jnp.int32, (TB, E), 1)

    @pl.when(p == 0)
    def _pass0():
        l = jnp.dot(x_ref[...], wg_ref[...], preferred_element_type=jnp.float32)
        l = l + bg_ref[0, :]
        m1 = jnp.max(l, axis=1, keepdims=True)
        i1 = jnp.min(jnp.where(l == m1, iota8, E), axis=1, keepdims=True)
        l2 = jnp.where(iota8 == i1, -jnp.inf, l)
        m2 = jnp.max(l2, axis=1, keepdims=True)
        i2 = jnp.min(jnp.where(l2 == m2, iota8, E), axis=1, keepdims=True)
        s1 = 1.0 / (1.0 + jnp.exp(m2 - m1))
        sl = pl.ds(t * TB, TB)
        si1[sl, :] = jnp.broadcast_to(i1, (TB, E))
        si2[sl, :] = jnp.broadcast_to(i2, (TB, E))
        ss1[sl, :] = jnp.broadcast_to(s1, (TB, E))

        @pl.when(t == 0)
        def _():
            scnt[...] = jnp.zeros((1, E), jnp.float32)

        oh = ((i1 == iota8) | (i2 == iota8)).astype(jnp.float32)
        scnt[...] += jnp.sum(oh, axis=0, keepdims=True)

    @pl.when(p == 1)
    def _pass1():
        @pl.when(t == 0)
        def _():
            cnt = scnt[...]
            cnti = cnt.astype(jnp.int32)
            padc = (((cnti + (BLK - 1)) // BLK) * BLK).astype(jnp.float32)
            r8 = lax.broadcasted_iota(jnp.int32, (E, E), 0)
            c8 = lax.broadcasted_iota(jnp.int32, (E, E), 1)
            U = (r8 <= c8).astype(jnp.float32)
            P88 = jnp.broadcast_to(padc, (E, E))
            ends = jnp.dot(P88, U, precision=_HIGH)[0:1, :]
            offs = ends - padc
            soff[...] = offs
            srun[...] = jnp.zeros((1, E), jnp.float32)
            eye = (r8 == c8).astype(jnp.float32)
            tdot = lambda a, b: lax.dot_general(
                a, b, (((1,), (1,)), ((), ())), precision=_HIGH)
            ends_c = tdot(eye, ends)
            offs_c = tdot(eye, offs)
            cnt_c = tdot(eye, cnt)
            bs = (lax.broadcasted_iota(jnp.int32, (E, 64), 1) * BLK
                  ).astype(jnp.float32)
            emap = jnp.sum((jnp.broadcast_to(ends_c, (E, 64)) <= bs)
                           .astype(jnp.int32), axis=0, keepdims=True)
            emap = jnp.minimum(emap, E - 1)
            inseg = ((bs >= offs_c) & (bs < offs_c + cnt_c)).astype(jnp.int32)
            act = jnp.sum(inseg, axis=0, keepdims=True)
            meta_ref[...] = jnp.concatenate([emap, act], axis=0)

        sl = pl.ds(t * TB, TB)
        i1 = si1[sl, :]
        i2 = si2[sl, :]
        oh1 = (i1 == iota8).astype(jnp.float32)
        oh2 = (i2 == iota8).astype(jnp.float32)
        oh = oh1 + oh2
        rr = lax.broadcasted_iota(jnp.int32, (TB, TB), 0)
        cc = lax.broadcasted_iota(jnp.int32, (TB, TB), 1)
        S = (cc < rr).astype(jnp.float32)
        C = jnp.dot(S, oh)
        A = C + soff[...] + srun[...]
        d0_ref[...] = jnp.sum(oh1 * A, axis=1, keepdims=True).astype(jnp.int32)
        d1_ref[...] = jnp.sum(oh2 * A, axis=1, keepdims=True).astype(jnp.int32)
        s1 = ss1[sl, 0:1]
        g1_ref[...] = jnp.broadcast_to(s1, (TB, 16))
        g2_ref[...] = jnp.broadcast_to(1.0 - s1, (TB, 16))
        srun[...] += jnp.sum(oh, axis=0, keepdims=True)


_gate_call = pl.pallas_call(
    _gate_body,
    grid=(2, NTB),
    in_specs=[
        pl.BlockSpec((TB, D), lambda p, t: (jnp.where(p == 0, t, NTB - 1), 0)),
        pl.BlockSpec((D, E), lambda p, t: (0, 0)),
        pl.BlockSpec((1, E), lambda p, t: (0, 0)),
    ],
    out_specs=[
        pl.BlockSpec((TB, 1), lambda p, t: (t, 0)),
        pl.BlockSpec((TB, 1), lambda p, t: (t, 0)),
        pl.BlockSpec((TB, 16), lambda p, t: (t, 0)),
        pl.BlockSpec((TB, 16), lambda p, t: (t, 0)),
        pl.BlockSpec((2, 64), lambda p, t: (0, 0)),
    ],
    out_shape=[
        jax.ShapeDtypeStruct((T, 1), jnp.int32),
        jax.ShapeDtypeStruct((T, 1), jnp.int32),
        jax.ShapeDtypeStruct((T, 16), jnp.float32),
        jax.ShapeDtypeStruct((T, 16), jnp.float32),
        jax.ShapeDtypeStruct((2, 64), jnp.int32),
    ],
    scratch_shapes=[
        pltpu.VMEM((T, E), jnp.int32),
        pltpu.VMEM((T, E), jnp.int32),
        pltpu.VMEM((T, E), jnp.float32),
        pltpu.VMEM((1, E), jnp.float32),
        pltpu.VMEM((1, E), jnp.float32),
        pltpu.VMEM((1, E), jnp.float32),
    ],
)


def _dispatch_body(x_hbm, d0_hbm, d1_hbm, xs_hbm, i0_v, i1_v, rows_v,
                   semr, sem0, sem1):
    w = lax.axis_index("s") * NC + lax.axis_index("c")
    cr = pltpu.async_copy(x_hbm.at[pl.ds(w * TPW, TPW)], rows_v, semr)
    pltpu.sync_copy(d0_hbm.at[w], i0_v)
    pltpu.sync_copy(d1_hbm.at[w], i1_v)
    cr.wait()
    c0 = pltpu.async_copy(rows_v, xs_hbm.at[i0_v], sem0)
    c1 = pltpu.async_copy(rows_v, xs_hbm.at[i1_v], sem1)
    c0.wait()
    c1.wait()


@functools.cache
def _dispatch_call():
    return pl.kernel(
        _dispatch_body,
        out_type=jax.ShapeDtypeStruct((SLOTS, D), jnp.float32),
        mesh=plsc.VectorSubcoreMesh(core_axis_name="c", subcore_axis_name="s"),
        scratch_types=[
            pltpu.VMEM((TPW,), jnp.int32),
            pltpu.VMEM((TPW,), jnp.int32),
            pltpu.VMEM((TPW, D), jnp.float32),
            pltpu.SemaphoreType.DMA,
            pltpu.SemaphoreType.DMA,
            pltpu.SemaphoreType.DMA,
        ],
    )


def _ffn_body(emap, act, xs_ref, w1_ref, b1_ref, w2_ref, b2_ref, out_ref):
    b = pl.program_id(0)

    @pl.when(act[b] == 1)
    def _():
        x = xs_ref[...]
        for f in range(F // FC):
            sl = slice(f * FC, (f + 1) * FC)
            h = jnp.dot(x, w1_ref[0][:, sl], preferred_element_type=jnp.float32)
            h = jnp.maximum(h + b1_ref[0, 0, sl], 0.0)
            p = jnp.dot(h, w2_ref[0][sl, :], preferred_element_type=jnp.float32)
            if f == 0:
                out_ref[...] = p + b2_ref[0, 0, :]
            else:
                out_ref[...] += p


_ffn_call = pl.pallas_call(
    _ffn_body,
    grid_spec=pltpu.PrefetchScalarGridSpec(
        num_scalar_prefetch=2,
        grid=(NBLK,),
        in_specs=[
            pl.BlockSpec((BLK, D), lambda b, em, ac: (b, 0)),
            pl.BlockSpec((1, D, F), lambda b, em, ac: (em[b], 0, 0)),
            pl.BlockSpec((1, 1, F), lambda b, em, ac: (em[b], 0, 0)),
            pl.BlockSpec((1, F, D), lambda b, em, ac: (em[b], 0, 0)),
            pl.BlockSpec((1, 1, D), lambda b, em, ac: (em[b], 0, 0)),
        ],
        out_specs=pl.BlockSpec((BLK, D), lambda b, em, ac: (b, 0)),
    ),
    out_shape=jax.ShapeDtypeStruct((SLOTS, D), jnp.float32),
)


QT = 16
NQ = TPW // QT


def _combine_body(ys_hbm, d0_hbm, d1_hbm, g0_hbm, g1_hbm, out_hbm,
                  i0_all, i1_all, g0_v, g1_v, y0_b, y1_b, ob_b,
                  sA, sB, soA, soB):
    w = lax.axis_index("s") * NC + lax.axis_index("c")
    pltpu.sync_copy(d0_hbm.at[w], i0_all)
    pltpu.sync_copy(d1_hbm.at[w], i1_all)
    pltpu.sync_copy(g0_hbm.at[pl.ds(w * TPW, TPW)], g0_v)
    pltpu.sync_copy(g1_hbm.at[pl.ds(w * TPW, TPW)], g1_v)
    sems = (sA, sB)
    osems = (soA, soB)

    def issue(q):
        bb = q % 2
        idx0 = i0_all[pl.ds(q * QT, QT)]
        idx1 = i1_all[pl.ds(q * QT, QT)]
        c0 = pltpu.async_copy(ys_hbm.at[idx0], y0_b.at[bb], sems[bb])
        c1 = pltpu.async_copy(ys_hbm.at[idx1], y1_b.at[bb], sems[bb])
        return (c0, c1)

    cps = [None] * NQ
    ocps = [None] * NQ
    cps[0] = issue(0)
    for q in range(NQ):
        if q + 1 < NQ:
            cps[q + 1] = issue(q + 1)
        cps[q][0].wait()
        cps[q][1].wait()
        if q >= 2:
            ocps[q - 2].wait()
        bb = q % 2
        base = q * QT

        def tok(j, carry, bb=bb, base=base):
            a = g0_v[base + j, :]
            b = g1_v[base + j, :]
            for v in range(D // 16):
                sl = pl.ds(v * 16, 16)
                ob_b[bb, j, sl] = a * y0_b[bb, j, sl] + b * y1_b[bb, j, sl]
            return carry

        lax.fori_loop(0, QT, tok, 0)
        ocps[q] = pltpu.async_copy(
            ob_b.at[bb], out_hbm.at[pl.ds(w * TPW + base, QT)], osems[bb])
    ocps[NQ - 2].wait()
    ocps[NQ - 1].wait()


@functools.cache
def _combine_call():
    return pl.kernel(
        _combine_body,
        out_type=jax.ShapeDtypeStruct((T, D), jnp.float32),
        mesh=plsc.VectorSubcoreMesh(core_axis_name="c", subcore_axis_name="s"),
        scratch_types=[
            pltpu.VMEM((TPW,), jnp.int32),
            pltpu.VMEM((TPW,), jnp.int32),
            pltpu.VMEM((TPW, 16), jnp.float32),
            pltpu.VMEM((TPW, 16), jnp.float32),
            pltpu.VMEM((2, QT, D), jnp.float32),
            pltpu.VMEM((2, QT, D), jnp.float32),
            pltpu.VMEM((2, QT, D), jnp.float32),
            pltpu.SemaphoreType.DMA,
            pltpu.SemaphoreType.DMA,
            pltpu.SemaphoreType.DMA,
            pltpu.SemaphoreType.DMA,
        ],
    )


def kernel(moe_inp, original_shape, total_experts, top_k, layer_idx,
           Wg, bg, W1, b1, W2, b2):
    x = moe_inp
    d0c, d1c, g1r, g2r, meta = _gate_call(x, Wg, bg.reshape(1, E))
    d0 = d0c.reshape(NW, TPW)
    d1 = d1c.reshape(NW, TPW)
    emap = meta[0, :NBLK]
    act = meta[1, :NBLK]

    xs = _dispatch_call()(x, d0, d1)
    ys = _ffn_call(emap, act, xs, W1, b1.reshape(E, 1, F),
                   W2, b2.reshape(E, 1, D))
    out = _combine_call()(ys, d0, d1, g1r, g2r)
    return out


# speedup vs baseline: 1.6722x; 1.0290x over previous
import functools

import jax
import jax.numpy as jnp
from jax import lax
from jax.experimental import pallas as pl
from jax.experimental.pallas import tpu as pltpu
from jax.experimental.pallas import tpu_sc as plsc

E = 8
D = 1024
F = 2048
K = 2
T = 2048

BLK = 512
NBLK = 16
SLOTS = NBLK * BLK
NC, NS = 2, 16
NW = NC * NS
TPW = T // NW
HALF = TPW // 2
FC = 512
TB = 512


NTB = T // TB
_HIGH = jax.lax.Precision.HIGHEST


def _gate_body(x_ref, wg_ref, bg_ref,
               d0_ref, d1_ref, g1_ref, g2_ref, meta_ref,
               si1, si2, ss1, scnt, srun, soff):
    p = pl.program_id(0)
    t = pl.program_id(1)
    iota8 = lax.broadcasted_iota(jnp.int32, (TB, E), 1)

    @pl.when(p == 0)
    def _pass0():
        l = jnp.dot(x_ref[...], wg_ref[...], preferred_element_type=jnp.float32)
        l = l + bg_ref[0, :]
        m1 = jnp.max(l, axis=1, keepdims=True)
        i1 = jnp.min(jnp.where(l == m1, iota8, E), axis=1, keepdims=True)
        l2 = jnp.where(iota8 == i1, -jnp.inf, l)
        m2 = jnp.max(l2, axis=1, keepdims=True)
        i2 = jnp.min(jnp.where(l2 == m2, iota8, E), axis=1, keepdims=True)
        s1 = 1.0 / (1.0 + jnp.exp(m2 - m1))
        sl = pl.ds(t * TB, TB)
        si1[sl, :] = jnp.broadcast_to(i1, (TB, E))
        si2[sl, :] = jnp.broadcast_to(i2, (TB, E))
        ss1[sl, :] = jnp.broadcast_to(s1, (TB, E))

        @pl.when(t == 0)
        def _():
            scnt[...] = jnp.zeros((1, E), jnp.float32)

        oh = ((i1 == iota8) | (i2 == iota8)).astype(jnp.float32)
        scnt[...] += jnp.sum(oh, axis=0, keepdims=True)

    @pl.when(p == 1)
    def _pass1():
        @pl.when(t == 0)
        def _():
            cnt = scnt[...]
            cnti = cnt.astype(jnp.int32)
            padc = (((cnti + (BLK - 1)) // BLK) * BLK).astype(jnp.float32)
            r8 = lax.broadcasted_iota(jnp.int32, (E, E), 0)
            c8 = lax.broadcasted_iota(jnp.int32, (E, E), 1)
            U = (r8 <= c8).astype(jnp.float32)
            P88 = jnp.broadcast_to(padc, (E, E))
            ends = jnp.dot(P88, U, precision=_HIGH)[0:1, :]
            offs = ends - padc
            soff[...] = offs
            srun[...] = jnp.zeros((1, E), jnp.float32)
            eye = (r8 == c8).astype(jnp.float32)
            tdot = lambda a, b: lax.dot_general(
                a, b, (((1,), (1,)), ((), ())), precision=_HIGH)
            ends_c = tdot(eye, ends)
            offs_c = tdot(eye, offs)
            cnt_c = tdot(eye, cnt)
            bs = (lax.broadcasted_iota(jnp.int32, (E, 64), 1) * BLK
                  ).astype(jnp.float32)
            emap = jnp.sum((jnp.broadcast_to(ends_c, (E, 64)) <= bs)
                           .astype(jnp.int32), axis=0, keepdims=True)
            emap = jnp.minimum(emap, E - 1)
            inseg = ((bs >= offs_c) & (bs < offs_c + cnt_c)).astype(jnp.int32)
            act = jnp.sum(inseg, axis=0, keepdims=True)
            meta_ref[...] = jnp.concatenate([emap, act], axis=0)

        sl = pl.ds(t * TB, TB)
        i1 = si1[sl, :]
        i2 = si2[sl, :]
        oh1 = (i1 == iota8).astype(jnp.float32)
        oh2 = (i2 == iota8).astype(jnp.float32)
        oh = oh1 + oh2
        rr = lax.broadcasted_iota(jnp.int32, (TB, TB), 0)
        cc = lax.broadcasted_iota(jnp.int32, (TB, TB), 1)
        S = (cc < rr).astype(jnp.float32)
        C = jnp.dot(S, oh)
        A = C + soff[...] + srun[...]
        d0_ref[...] = jnp.sum(oh1 * A, axis=1, keepdims=True).astype(jnp.int32)
        d1_ref[...] = jnp.sum(oh2 * A, axis=1, keepdims=True).astype(jnp.int32)
        s1 = ss1[sl, 0:1]
        g1_ref[...] = jnp.broadcast_to(s1, (TB, 16))
        g2_ref[...] = jnp.broadcast_to(1.0 - s1, (TB, 16))
        srun[...] += jnp.sum(oh, axis=0, keepdims=True)


_gate_call = pl.pallas_call(
    _gate_body,
    grid=(2, NTB),
    in_specs=[
        pl.BlockSpec((TB, D), lambda p, t: (jnp.where(p == 0, t, NTB - 1), 0)),
        pl.BlockSpec((D, E), lambda p, t: (0, 0)),
        pl.BlockSpec((1, E), lambda p, t: (0, 0)),
    ],
    out_specs=[
        pl.BlockSpec((TB, 1), lambda p, t: (t, 0)),
        pl.BlockSpec((TB, 1), lambda p, t: (t, 0)),
        pl.BlockSpec((TB, 16), lambda p, t: (t, 0)),
        pl.BlockSpec((TB, 16), lambda p, t: (t, 0)),
        pl.BlockSpec((2, 64), lambda p, t: (0, 0)),
    ],
    out_shape=[
        jax.ShapeDtypeStruct((T, 1), jnp.int32),
        jax.ShapeDtypeStruct((T, 1), jnp.int32),
        jax.ShapeDtypeStruct((T, 16), jnp.float32),
        jax.ShapeDtypeStruct((T, 16), jnp.float32),
        jax.ShapeDtypeStruct((2, 64), jnp.int32),
    ],
    scratch_shapes=[
        pltpu.VMEM((T, E), jnp.int32),
        pltpu.VMEM((T, E), jnp.int32),
        pltpu.VMEM((T, E), jnp.float32),
        pltpu.VMEM((1, E), jnp.float32),
        pltpu.VMEM((1, E), jnp.float32),
        pltpu.VMEM((1, E), jnp.float32),
    ],
)


def _dispatch_body(x_hbm, d0_hbm, d1_hbm, xs_hbm, i0_v, i1_v, rows_v,
                   semr, sem0, sem1):
    w = lax.axis_index("s") * NC + lax.axis_index("c")
    cr = pltpu.async_copy(x_hbm.at[pl.ds(w * TPW, TPW)], rows_v, semr)
    pltpu.sync_copy(d0_hbm.at[w], i0_v)
    pltpu.sync_copy(d1_hbm.at[w], i1_v)
    cr.wait()
    c0 = pltpu.async_copy(rows_v, xs_hbm.at[i0_v], sem0)
    c1 = pltpu.async_copy(rows_v, xs_hbm.at[i1_v], sem1)
    c0.wait()
    c1.wait()


@functools.cache
def _dispatch_call():
    return pl.kernel(
        _dispatch_body,
        out_type=jax.ShapeDtypeStruct((SLOTS, D), jnp.float32),
        mesh=plsc.VectorSubcoreMesh(core_axis_name="c", subcore_axis_name="s"),
        scratch_types=[
            pltpu.VMEM((TPW,), jnp.int32),
            pltpu.VMEM((TPW,), jnp.int32),
            pltpu.VMEM((TPW, D), jnp.float32),
            pltpu.SemaphoreType.DMA,
            pltpu.SemaphoreType.DMA,
            pltpu.SemaphoreType.DMA,
        ],
    )


def _ffn_body(emap, act, xs_ref, w1_ref, b1_ref, w2_ref, b2_ref, out_ref):
    b = pl.program_id(0)

    @pl.when(act[b] == 1)
    def _():
        x = xs_ref[...]
        for f in range(F // FC):
            sl = slice(f * FC, (f + 1) * FC)
            h = jnp.dot(x, w1_ref[0][:, sl], preferred_element_type=jnp.float32)
            h = jnp.maximum(h + b1_ref[0, 0, sl], 0.0)
            p = jnp.dot(h, w2_ref[0][sl, :], preferred_element_type=jnp.float32)
            if f == 0:
                out_ref[...] = p + b2_ref[0, 0, :]
            else:
                out_ref[...] += p


_ffn_call = pl.pallas_call(
    _ffn_body,
    grid_spec=pltpu.PrefetchScalarGridSpec(
        num_scalar_prefetch=2,
        grid=(NBLK,),
        in_specs=[
            pl.BlockSpec((BLK, D), lambda b, em, ac: (b, 0)),
            pl.BlockSpec((1, D, F), lambda b, em, ac: (em[b], 0, 0)),
            pl.BlockSpec((1, 1, F), lambda b, em, ac: (em[b], 0, 0)),
            pl.BlockSpec((1, F, D), lambda b, em, ac: (em[b], 0, 0)),
            pl.BlockSpec((1, 1, D), lambda b, em, ac: (em[b], 0, 0)),
        ],
        out_specs=pl.BlockSpec((BLK, D), lambda b, em, ac: (b, 0)),
    ),
    out_shape=jax.ShapeDtypeStruct((SLOTS, D), jnp.float32),
)


QT = 16
NQ = TPW // QT


def _combine_body(ys_hbm, d0_hbm, d1_hbm, g0_hbm, g1_hbm, out_hbm,
                  i0_all, i1_all, g0_v, g1_v, y0_b, y1_b, ob_b,
                  sA, sB, soA, soB):
    w = lax.axis_index("s") * NC + lax.axis_index("c")
    pltpu.sync_copy(d0_hbm.at[w], i0_all)
    pltpu.sync_copy(d1_hbm.at[w], i1_all)
    pltpu.sync_copy(g0_hbm.at[pl.ds(w * TPW, TPW)], g0_v)
    pltpu.sync_copy(g1_hbm.at[pl.ds(w * TPW, TPW)], g1_v)
    sems = (sA, sB)
    osems = (soA, soB)

    def issue(q):
        bb = q % 2
        idx0 = i0_all[pl.ds(q * QT, QT)]
        idx1 = i1_all[pl.ds(q * QT, QT)]
        c0 = pltpu.async_copy(ys_hbm.at[idx0], y0_b.at[bb], sems[bb])
        c1 = pltpu.async_copy(ys_hbm.at[idx1], y1_b.at[bb], sems[bb])
        return (c0, c1)

    cps = [None] * NQ
    ocps = [None] * NQ
    cps[0] = issue(0)
    for q in range(NQ):
        if q + 1 < NQ:
            cps[q + 1] = issue(q + 1)
        cps[q][0].wait()
        cps[q][1].wait()
        if q >= 2:
            ocps[q - 2].wait()
        bb = q % 2
        base = q * QT

        def tok(j, carry, bb=bb, base=base):
            a = g0_v[base + j, :]
            b = g1_v[base + j, :]
            for v in range(D // 16):
                sl = pl.ds(v * 16, 16)
                ob_b[bb, j, sl] = a * y0_b[bb, j, sl] + b * y1_b[bb, j, sl]
            return carry

        lax.fori_loop(0, QT, tok, 0)
        ocps[q] = pltpu.async_copy(
            ob_b.at[bb], out_hbm.at[pl.ds(w * TPW + base, QT)], osems[bb])
    ocps[NQ - 2].wait()
    ocps[NQ - 1].wait()


@functools.cache
def _combine_call():
    return pl.kernel(
        _combine_body,
        out_type=jax.ShapeDtypeStruct((T, D), jnp.float32),
        mesh=plsc.VectorSubcoreMesh(core_axis_name="c", subcore_axis_name="s"),
        scratch_types=[
            pltpu.VMEM((TPW,), jnp.int32),
            pltpu.VMEM((TPW,), jnp.int32),
            pltpu.VMEM((TPW, 16), jnp.float32),
            pltpu.VMEM((TPW, 16), jnp.float32),
            pltpu.VMEM((2, QT, D), jnp.float32),
            pltpu.VMEM((2, QT, D), jnp.float32),
            pltpu.VMEM((2, QT, D), jnp.float32),
            pltpu.SemaphoreType.DMA,
            pltpu.SemaphoreType.DMA,
            pltpu.SemaphoreType.DMA,
            pltpu.SemaphoreType.DMA,
        ],
    )


def kernel(moe_inp, original_shape, total_experts, top_k, layer_idx,
           Wg, bg, W1, b1, W2, b2):
    x = moe_inp
    d0c, d1c, g1r, g2r, meta = _gate_call(x, Wg, bg.reshape(1, E))
    d0 = d0c.reshape(NW, TPW)
    d1 = d1c.reshape(NW, TPW)
    emap = meta[0, :NBLK]
    act = meta[1, :NBLK]

    xs = _dispatch_call()(x, d0, d1)
    ys = _ffn_call(emap, act, xs, W1, b1.reshape(E, 1, F),
                   W2, b2.reshape(E, 1, D))
    out = _combine_call()(ys, d0, d1, g1r, g2r)
    return out


# speedup vs baseline: 1.6930x; 1.0125x over previous
import functools

import jax
import jax.numpy as jnp
from jax import lax
from jax.experimental import pallas as pl
from jax.experimental.pallas import tpu as pltpu
from jax.experimental.pallas import tpu_sc as plsc

E = 8
D = 1024
F = 2048
K = 2
T = 2048

BLK = 512
NBLK = 16
SLOTS = NBLK * BLK
NC, NS = 2, 16
NW = NC * NS
TPW = T // NW
HALF = TPW // 2
FC = 512
TB = 1024


NTB = T // TB
_HIGH = jax.lax.Precision.HIGHEST


def _gate_body(x_ref, wg_ref, bg_ref,
               d0_ref, d1_ref, g1_ref, g2_ref, meta_ref,
               si1, si2, ss1, scnt, srun, soff):
    p = pl.program_id(0)
    t = pl.program_id(1)
    iota8 = lax.broadcasted_iota(jnp.int32, (TB, E), 1)

    @pl.when(p == 0)
    def _pass0():
        l = jnp.dot(x_ref[...], wg_ref[...], preferred_element_type=jnp.float32)
        l = l + bg_ref[0, :]
        m1 = jnp.max(l, axis=1, keepdims=True)
        i1 = jnp.min(jnp.where(l == m1, iota8, E), axis=1, keepdims=True)
        l2 = jnp.where(iota8 == i1, -jnp.inf, l)
        m2 = jnp.max(l2, axis=1, keepdims=True)
        i2 = jnp.min(jnp.where(l2 == m2, iota8, E), axis=1, keepdims=True)
        s1 = 1.0 / (1.0 + jnp.exp(m2 - m1))
        sl = pl.ds(t * TB, TB)
        si1[sl, :] = jnp.broadcast_to(i1, (TB, E))
        si2[sl, :] = jnp.broadcast_to(i2, (TB, E))
        ss1[sl, :] = jnp.broadcast_to(s1, (TB, E))

        @pl.when(t == 0)
        def _():
            scnt[...] = jnp.zeros((1, E), jnp.float32)

        oh = ((i1 == iota8) | (i2 == iota8)).astype(jnp.float32)
        scnt[...] += jnp.sum(oh, axis=0, keepdims=True)

    @pl.when(p == 1)
    def _pass1():
        @pl.when(t == 0)
        def _():
            cnt = scnt[...]
            cnti = cnt.astype(jnp.int32)
            padc = (((cnti + (BLK - 1)) // BLK) * BLK).astype(jnp.float32)
            r8 = lax.broadcasted_iota(jnp.int32, (E, E), 0)
            c8 = lax.broadcasted_iota(jnp.int32, (E, E), 1)
            U = (r8 <= c8).astype(jnp.float32)
            P88 = jnp.broadcast_to(padc, (E, E))
            ends = jnp.dot(P88, U, precision=_HIGH)[0:1, :]
            offs = ends - padc
            soff[...] = offs
            srun[...] = jnp.zeros((1, E), jnp.float32)
            eye = (r8 == c8).astype(jnp.float32)
            tdot = lambda a, b: lax.dot_general(
                a, b, (((1,), (1,)), ((), ())), precision=_HIGH)
            ends_c = tdot(eye, ends)
            offs_c = tdot(eye, offs)
            cnt_c = tdot(eye, cnt)
            bs = (lax.broadcasted_iota(jnp.int32, (E, 64), 1) * BLK
                  ).astype(jnp.float32)
            emap = jnp.sum((jnp.broadcast_to(ends_c, (E, 64)) <= bs)
                           .astype(jnp.int32), axis=0, keepdims=True)
            emap = jnp.minimum(emap, E - 1)
            inseg = ((bs >= offs_c) & (bs < offs_c + cnt_c)).astype(jnp.int32)
            act = jnp.sum(inseg, axis=0, keepdims=True)
            meta_ref[...] = jnp.concatenate([emap, act], axis=0)

        sl = pl.ds(t * TB, TB)
        i1 = si1[sl, :]
        i2 = si2[sl, :]
        oh1 = (i1 == iota8).astype(jnp.float32)
        oh2 = (i2 == iota8).astype(jnp.float32)
        oh = oh1 + oh2
        rr = lax.broadcasted_iota(jnp.int32, (TB, TB), 0)
        cc = lax.broadcasted_iota(jnp.int32, (TB, TB), 1)
        S = (cc < rr).astype(jnp.float32)
        C = jnp.dot(S, oh)
        A = C + soff[...] + srun[...]
        d0_ref[...] = jnp.sum(oh1 * A, axis=1, keepdims=True).astype(jnp.int32)
        d1_ref[...] = jnp.sum(oh2 * A, axis=1, keepdims=True).astype(jnp.int32)
        s1 = ss1[sl, 0:1]
        g1_ref[...] = jnp.broadcast_to(s1, (TB, 16))
        g2_ref[...] = jnp.broadcast_to(1.0 - s1, (TB, 16))
        srun[...] += jnp.sum(oh, axis=0, keepdims=True)


_gate_call = pl.pallas_call(
    _gate_body,
    grid=(2, NTB),
    in_specs=[
        pl.BlockSpec((TB, D), lambda p, t: (jnp.where(p == 0, t, NTB - 1), 0)),
        pl.BlockSpec((D, E), lambda p, t: (0, 0)),
        pl.BlockSpec((1, E), lambda p, t: (0, 0)),
    ],
    out_specs=[
        pl.BlockSpec((TB, 1), lambda p, t: (t, 0)),
        pl.BlockSpec((TB, 1), lambda p, t: (t, 0)),
        pl.BlockSpec((TB, 16), lambda p, t: (t, 0)),
        pl.BlockSpec((TB, 16), lambda p, t: (t, 0)),
        pl.BlockSpec((2, 64), lambda p, t: (0, 0)),
    ],
    out_shape=[
        jax.ShapeDtypeStruct((T, 1), jnp.int32),
        jax.ShapeDtypeStruct((T, 1), jnp.int32),
        jax.ShapeDtypeStruct((T, 16), jnp.float32),
        jax.ShapeDtypeStruct((T, 16), jnp.float32),
        jax.ShapeDtypeStruct((2, 64), jnp.int32),
    ],
    scratch_shapes=[
        pltpu.VMEM((T, E), jnp.int32),
        pltpu.VMEM((T, E), jnp.int32),
        pltpu.VMEM((T, E), jnp.float32),
        pltpu.VMEM((1, E), jnp.float32),
        pltpu.VMEM((1, E), jnp.float32),
        pltpu.VMEM((1, E), jnp.float32),
    ],
)


def _dispatch_body(x_hbm, d0_hbm, d1_hbm, xs_hbm, i0_v, i1_v, rows_v,
                   semr, sem0, sem1):
    w = lax.axis_index("s") * NC + lax.axis_index("c")
    cr = pltpu.async_copy(x_hbm.at[pl.ds(w * TPW, TPW)], rows_v, semr)
    pltpu.sync_copy(d0_hbm.at[w], i0_v)
    pltpu.sync_copy(d1_hbm.at[w], i1_v)
    cr.wait()
    c0 = pltpu.async_copy(rows_v, xs_hbm.at[i0_v], sem0)
    c1 = pltpu.async_copy(rows_v, xs_hbm.at[i1_v], sem1)
    c0.wait()
    c1.wait()


@functools.cache
def _dispatch_call():
    return pl.kernel(
        _dispatch_body,
        out_type=jax.ShapeDtypeStruct((SLOTS, D), jnp.float32),
        mesh=plsc.VectorSubcoreMesh(core_axis_name="c", subcore_axis_name="s"),
        scratch_types=[
            pltpu.VMEM((TPW,), jnp.int32),
            pltpu.VMEM((TPW,), jnp.int32),
            pltpu.VMEM((TPW, D), jnp.float32),
            pltpu.SemaphoreType.DMA,
            pltpu.SemaphoreType.DMA,
            pltpu.SemaphoreType.DMA,
        ],
    )


def _ffn_body(emap, act, xs_ref, w1_ref, b1_ref, w2_ref, b2_ref, out_ref):
    b = pl.program_id(0)

    @pl.when(act[b] == 1)
    def _():
        x = xs_ref[...]
        for f in range(F // FC):
            sl = slice(f * FC, (f + 1) * FC)
            h = jnp.dot(x, w1_ref[0][:, sl], preferred_element_type=jnp.float32)
            h = jnp.maximum(h + b1_ref[0, 0, sl], 0.0)
            p = jnp.dot(h, w2_ref[0][sl, :], preferred_element_type=jnp.float32)
            if f == 0:
                out_ref[...] = p + b2_ref[0, 0, :]
            else:
                out_ref[...] += p


_ffn_call = pl.pallas_call(
    _ffn_body,
    grid_spec=pltpu.PrefetchScalarGridSpec(
        num_scalar_prefetch=2,
        grid=(NBLK,),
        in_specs=[
            pl.BlockSpec((BLK, D), lambda b, em, ac: (b, 0)),
            pl.BlockSpec((1, D, F), lambda b, em, ac: (em[b], 0, 0)),
            pl.BlockSpec((1, 1, F), lambda b, em, ac: (em[b], 0, 0)),
            pl.BlockSpec((1, F, D), lambda b, em, ac: (em[b], 0, 0)),
            pl.BlockSpec((1, 1, D), lambda b, em, ac: (em[b], 0, 0)),
        ],
        out_specs=pl.BlockSpec((BLK, D), lambda b, em, ac: (b, 0)),
    ),
    out_shape=jax.ShapeDtypeStruct((SLOTS, D), jnp.float32),
)


QT = 16
NQ = TPW // QT


def _combine_body(ys_hbm, d0_hbm, d1_hbm, g0_hbm, g1_hbm, out_hbm,
                  i0_all, i1_all, g0_v, g1_v, y0_b, y1_b, ob_b,
                  sA, sB, soA, soB):
    w = lax.axis_index("s") * NC + lax.axis_index("c")
    pltpu.sync_copy(d0_hbm.at[w], i0_all)
    pltpu.sync_copy(d1_hbm.at[w], i1_all)
    pltpu.sync_copy(g0_hbm.at[pl.ds(w * TPW, TPW)], g0_v)
    pltpu.sync_copy(g1_hbm.at[pl.ds(w * TPW, TPW)], g1_v)
    sems = (sA, sB)
    osems = (soA, soB)

    def issue(q):
        bb = q % 2
        idx0 = i0_all[pl.ds(q * QT, QT)]
        idx1 = i1_all[pl.ds(q * QT, QT)]
        c0 = pltpu.async_copy(ys_hbm.at[idx0], y0_b.at[bb], sems[bb])
        c1 = pltpu.async_copy(ys_hbm.at[idx1], y1_b.at[bb], sems[bb])
        return (c0, c1)

    cps = [None] * NQ
    ocps = [None] * NQ
    cps[0] = issue(0)
    for q in range(NQ):
        if q + 1 < NQ:
            cps[q + 1] = issue(q + 1)
        cps[q][0].wait()
        cps[q][1].wait()
        if q >= 2:
            ocps[q - 2].wait()
        bb = q % 2
        base = q * QT

        def tok(j, carry, bb=bb, base=base):
            a = g0_v[base + j, :]
            b = g1_v[base + j, :]
            for v in range(D // 16):
                sl = pl.ds(v * 16, 16)
                ob_b[bb, j, sl] = a * y0_b[bb, j, sl] + b * y1_b[bb, j, sl]
            return carry

        lax.fori_loop(0, QT, tok, 0)
        ocps[q] = pltpu.async_copy(
            ob_b.at[bb], out_hbm.at[pl.ds(w * TPW + base, QT)], osems[bb])
    ocps[NQ - 2].wait()
    ocps[NQ - 1].wait()


@functools.cache
def _combine_call():
    return pl.kernel(
        _combine_body,
        out_type=jax.ShapeDtypeStruct((T, D), jnp.float32),
        mesh=plsc.VectorSubcoreMesh(core_axis_name="c", subcore_axis_name="s"),
        scratch_types=[
            pltpu.VMEM((TPW,), jnp.int32),
            pltpu.VMEM((TPW,), jnp.int32),
            pltpu.VMEM((TPW, 16), jnp.float32),
            pltpu.VMEM((TPW, 16), jnp.float32),
            pltpu.VMEM((2, QT, D), jnp.float32),
            pltpu.VMEM((2, QT, D), jnp.float32),
            pltpu.VMEM((2, QT, D), jnp.float32),
            pltpu.SemaphoreType.DMA,
            pltpu.SemaphoreType.DMA,
            pltpu.SemaphoreType.DMA,
            pltpu.SemaphoreType.DMA,
        ],
    )


def kernel(moe_inp, original_shape, total_experts, top_k, layer_idx,
           Wg, bg, W1, b1, W2, b2):
    x = moe_inp
    d0c, d1c, g1r, g2r, meta = _gate_call(x, Wg, bg.reshape(1, E))
    d0 = d0c.reshape(NW, TPW)
    d1 = d1c.reshape(NW, TPW)
    emap = meta[0, :NBLK]
    act = meta[1, :NBLK]

    xs = _dispatch_call()(x, d0, d1)
    ys = _ffn_call(emap, act, xs, W1, b1.reshape(E, 1, F),
                   W2, b2.reshape(E, 1, D))
    out = _combine_call()(ys, d0, d1, g1r, g2r)
    return out


# speedup vs baseline: 1.8338x; 1.0831x over previous
import functools

import jax
import jax.numpy as jnp
from jax import lax
from jax.experimental import pallas as pl
from jax.experimental.pallas import tpu as pltpu
from jax.experimental.pallas import tpu_sc as plsc

E = 8
D = 1024
F = 2048
K = 2
T = 2048

BLK = 768
NBLK = 14
SLOTS = NBLK * BLK
NC, NS = 2, 16
NW = NC * NS
TPW = T // NW
HALF = TPW // 2
FC = 512
TB = 1024


NTB = T // TB
_HIGH = jax.lax.Precision.HIGHEST


def _gate_body(x_ref, wg_ref, bg_ref,
               d0_ref, d1_ref, g1_ref, g2_ref, meta_ref,
               si1, si2, ss1, scnt, srun, soff):
    p = pl.program_id(0)
    t = pl.program_id(1)
    iota8 = lax.broadcasted_iota(jnp.int32, (TB, E), 1)

    @pl.when(p == 0)
    def _pass0():
        l = jnp.dot(x_ref[...], wg_ref[...], preferred_element_type=jnp.float32)
        l = l + bg_ref[0, :]
        m1 = jnp.max(l, axis=1, keepdims=True)
        i1 = jnp.min(jnp.where(l == m1, iota8, E), axis=1, keepdims=True)
        l2 = jnp.where(iota8 == i1, -jnp.inf, l)
        m2 = jnp.max(l2, axis=1, keepdims=True)
        i2 = jnp.min(jnp.where(l2 == m2, iota8, E), axis=1, keepdims=True)
        s1 = 1.0 / (1.0 + jnp.exp(m2 - m1))
        sl = pl.ds(t * TB, TB)
        si1[sl, :] = jnp.broadcast_to(i1, (TB, E))
        si2[sl, :] = jnp.broadcast_to(i2, (TB, E))
        ss1[sl, :] = jnp.broadcast_to(s1, (TB, E))

        @pl.when(t == 0)
        def _():
            scnt[...] = jnp.zeros((1, E), jnp.float32)

        oh = ((i1 == iota8) | (i2 == iota8)).astype(jnp.float32)
        scnt[...] += jnp.sum(oh, axis=0, keepdims=True)

    @pl.when(p == 1)
    def _pass1():
        @pl.when(t == 0)
        def _():
            cnt = scnt[...]
            cnti = cnt.astype(jnp.int32)
            padc = (((cnti + (BLK - 1)) // BLK) * BLK).astype(jnp.float32)
            r8 = lax.broadcasted_iota(jnp.int32, (E, E), 0)
            c8 = lax.broadcasted_iota(jnp.int32, (E, E), 1)
            U = (r8 <= c8).astype(jnp.float32)
            P88 = jnp.broadcast_to(padc, (E, E))
            ends = jnp.dot(P88, U, precision=_HIGH)[0:1, :]
            offs = ends - padc
            soff[...] = offs
            srun[...] = jnp.zeros((1, E), jnp.float32)
            eye = (r8 == c8).astype(jnp.float32)
            tdot = lambda a, b: lax.dot_general(
                a, b, (((1,), (1,)), ((), ())), precision=_HIGH)
            ends_c = tdot(eye, ends)
            offs_c = tdot(eye, offs)
            cnt_c = tdot(eye, cnt)
            bs = (lax.broadcasted_iota(jnp.int32, (E, 64), 1) * BLK
                  ).astype(jnp.float32)
            emap = jnp.sum((jnp.broadcast_to(ends_c, (E, 64)) <= bs)
                           .astype(jnp.int32), axis=0, keepdims=True)
            emap = jnp.minimum(emap, E - 1)
            inseg = ((bs >= offs_c) & (bs < offs_c + cnt_c)).astype(jnp.int32)
            act = jnp.sum(inseg, axis=0, keepdims=True)
            meta_ref[...] = jnp.concatenate([emap, act], axis=0)

        sl = pl.ds(t * TB, TB)
        i1 = si1[sl, :]
        i2 = si2[sl, :]
        oh1 = (i1 == iota8).astype(jnp.float32)
        oh2 = (i2 == iota8).astype(jnp.float32)
        oh = oh1 + oh2
        rr = lax.broadcasted_iota(jnp.int32, (TB, TB), 0)
        cc = lax.broadcasted_iota(jnp.int32, (TB, TB), 1)
        S = (cc < rr).astype(jnp.float32)
        C = jnp.dot(S, oh)
        A = C + soff[...] + srun[...]
        d0_ref[...] = jnp.sum(oh1 * A, axis=1, keepdims=True).astype(jnp.int32)
        d1_ref[...] = jnp.sum(oh2 * A, axis=1, keepdims=True).astype(jnp.int32)
        s1 = ss1[sl, 0:1]
        g1_ref[...] = jnp.broadcast_to(s1, (TB, 16))
        g2_ref[...] = jnp.broadcast_to(1.0 - s1, (TB, 16))
        srun[...] += jnp.sum(oh, axis=0, keepdims=True)


_gate_call = pl.pallas_call(
    _gate_body,
    grid=(2, NTB),
    in_specs=[
        pl.BlockSpec((TB, D), lambda p, t: (jnp.where(p == 0, t, NTB - 1), 0)),
        pl.BlockSpec((D, E), lambda p, t: (0, 0)),
        pl.BlockSpec((1, E), lambda p, t: (0, 0)),
    ],
    out_specs=[
        pl.BlockSpec((TB, 1), lambda p, t: (t, 0)),
        pl.BlockSpec((TB, 1), lambda p, t: (t, 0)),
        pl.BlockSpec((TB, 16), lambda p, t: (t, 0)),
        pl.BlockSpec((TB, 16), lambda p, t: (t, 0)),
        pl.BlockSpec((2, 64), lambda p, t: (0, 0)),
    ],
    out_shape=[
        jax.ShapeDtypeStruct((T, 1), jnp.int32),
        jax.ShapeDtypeStruct((T, 1), jnp.int32),
        jax.ShapeDtypeStruct((T, 16), jnp.float32),
        jax.ShapeDtypeStruct((T, 16), jnp.float32),
        jax.ShapeDtypeStruct((2, 64), jnp.int32),
    ],
    scratch_shapes=[
        pltpu.VMEM((T, E), jnp.int32),
        pltpu.VMEM((T, E), jnp.int32),
        pltpu.VMEM((T, E), jnp.float32),
        pltpu.VMEM((1, E), jnp.float32),
        pltpu.VMEM((1, E), jnp.float32),
        pltpu.VMEM((1, E), jnp.float32),
    ],
)


def _dispatch_body(x_hbm, d0_hbm, d1_hbm, xs_hbm, i0_v, i1_v, rows_v,
                   semr, sem0, sem1):
    w = lax.axis_index("s") * NC + lax.axis_index("c")
    cr = pltpu.async_copy(x_hbm.at[pl.ds(w * TPW, TPW)], rows_v, semr)
    pltpu.sync_copy(d0_hbm.at[w], i0_v)
    pltpu.sync_copy(d1_hbm.at[w], i1_v)
    cr.wait()
    c0 = pltpu.async_copy(rows_v, xs_hbm.at[i0_v], sem0)
    c1 = pltpu.async_copy(rows_v, xs_hbm.at[i1_v], sem1)
    c0.wait()
    c1.wait()


@functools.cache
def _dispatch_call():
    return pl.kernel(
        _dispatch_body,
        out_type=jax.ShapeDtypeStruct((SLOTS, D), jnp.float32),
        mesh=plsc.VectorSubcoreMesh(core_axis_name="c", subcore_axis_name="s"),
        scratch_types=[
            pltpu.VMEM((TPW,), jnp.int32),
            pltpu.VMEM((TPW,), jnp.int32),
            pltpu.VMEM((TPW, D), jnp.float32),
            pltpu.SemaphoreType.DMA,
            pltpu.SemaphoreType.DMA,
            pltpu.SemaphoreType.DMA,
        ],
    )


def _ffn_body(emap, act, xs_ref, w1_ref, b1_ref, w2_ref, b2_ref, out_ref):
    b = pl.program_id(0)

    @pl.when(act[b] == 1)
    def _():
        x = xs_ref[...]
        for f in range(F // FC):
            sl = slice(f * FC, (f + 1) * FC)
            h = jnp.dot(x, w1_ref[0][:, sl], preferred_element_type=jnp.float32)
            h = jnp.maximum(h + b1_ref[0, 0, sl], 0.0)
            p = jnp.dot(h, w2_ref[0][sl, :], preferred_element_type=jnp.float32)
            if f == 0:
                out_ref[...] = p + b2_ref[0, 0, :]
            else:
                out_ref[...] += p


_ffn_call = pl.pallas_call(
    _ffn_body,
    grid_spec=pltpu.PrefetchScalarGridSpec(
        num_scalar_prefetch=2,
        grid=(NBLK,),
        in_specs=[
            pl.BlockSpec((BLK, D),
                         lambda b, em, ac: (jnp.where(ac[b] == 1, b, 0), 0)),
            pl.BlockSpec((1, D, F), lambda b, em, ac: (em[b], 0, 0)),
            pl.BlockSpec((1, 1, F), lambda b, em, ac: (em[b], 0, 0)),
            pl.BlockSpec((1, F, D), lambda b, em, ac: (em[b], 0, 0)),
            pl.BlockSpec((1, 1, D), lambda b, em, ac: (em[b], 0, 0)),
        ],
        out_specs=pl.BlockSpec((BLK, D), lambda b, em, ac: (b, 0)),
    ),
    out_shape=jax.ShapeDtypeStruct((SLOTS, D), jnp.float32),
)


QT = 16
NQ = TPW // QT


def _combine_body(ys_hbm, d0_hbm, d1_hbm, g0_hbm, g1_hbm, out_hbm,
                  i0_all, i1_all, g0_v, g1_v, y0_b, y1_b, ob_b,
                  sA, sB, soA, soB):
    w = lax.axis_index("s") * NC + lax.axis_index("c")
    pltpu.sync_copy(d0_hbm.at[w], i0_all)
    pltpu.sync_copy(d1_hbm.at[w], i1_all)
    pltpu.sync_copy(g0_hbm.at[pl.ds(w * TPW, TPW)], g0_v)
    pltpu.sync_copy(g1_hbm.at[pl.ds(w * TPW, TPW)], g1_v)
    sems = (sA, sB)
    osems = (soA, soB)

    def issue(q):
        bb = q % 2
        idx0 = i0_all[pl.ds(q * QT, QT)]
        idx1 = i1_all[pl.ds(q * QT, QT)]
        c0 = pltpu.async_copy(ys_hbm.at[idx0], y0_b.at[bb], sems[bb])
        c1 = pltpu.async_copy(ys_hbm.at[idx1], y1_b.at[bb], sems[bb])
        return (c0, c1)

    cps = [None] * NQ
    ocps = [None] * NQ
    cps[0] = issue(0)
    for q in range(NQ):
        if q + 1 < NQ:
            cps[q + 1] = issue(q + 1)
        cps[q][0].wait()
        cps[q][1].wait()
        if q >= 2:
            ocps[q - 2].wait()
        bb = q % 2
        base = q * QT

        def tok(j, carry, bb=bb, base=base):
            a = g0_v[base + j, :]
            b = g1_v[base + j, :]
            for v in range(D // 16):
                sl = pl.ds(v * 16, 16)
                ob_b[bb, j, sl] = a * y0_b[bb, j, sl] + b * y1_b[bb, j, sl]
            return carry

        lax.fori_loop(0, QT, tok, 0)
        ocps[q] = pltpu.async_copy(
            ob_b.at[bb], out_hbm.at[pl.ds(w * TPW + base, QT)], osems[bb])
    ocps[NQ - 2].wait()
    ocps[NQ - 1].wait()


@functools.cache
def _combine_call():
    return pl.kernel(
        _combine_body,
        out_type=jax.ShapeDtypeStruct((T, D), jnp.float32),
        mesh=plsc.VectorSubcoreMesh(core_axis_name="c", subcore_axis_name="s"),
        scratch_types=[
            pltpu.VMEM((TPW,), jnp.int32),
            pltpu.VMEM((TPW,), jnp.int32),
            pltpu.VMEM((TPW, 16), jnp.float32),
            pltpu.VMEM((TPW, 16), jnp.float32),
            pltpu.VMEM((2, QT, D), jnp.float32),
            pltpu.VMEM((2, QT, D), jnp.float32),
            pltpu.VMEM((2, QT, D), jnp.float32),
            pltpu.SemaphoreType.DMA,
            pltpu.SemaphoreType.DMA,
            pltpu.SemaphoreType.DMA,
            pltpu.SemaphoreType.DMA,
        ],
    )


def kernel(moe_inp, original_shape, total_experts, top_k, layer_idx,
           Wg, bg, W1, b1, W2, b2):
    x = moe_inp
    d0c, d1c, g1r, g2r, meta = _gate_call(x, Wg, bg.reshape(1, E))
    d0 = d0c.reshape(NW, TPW)
    d1 = d1c.reshape(NW, TPW)
    emap = meta[0, :NBLK]
    act = meta[1, :NBLK]

    xs = _dispatch_call()(x, d0, d1)
    ys = _ffn_call(emap, act, xs, W1, b1.reshape(E, 1, F),
                   W2, b2.reshape(E, 1, D))
    out = _combine_call()(ys, d0, d1, g1r, g2r)
    return out
